# Initial kernel scaffold; baseline (speedup 1.0000x reference)
import jax
import jax.numpy as jnp
from jax.experimental import pallas as pl


def kernel(qid_table, uid_table, click_table, vid_table, pos_table, W_q, a_src_q, a_dst_q, b_q, W_u, a_src_u, a_dst_u, b_u, qid_edge_index, uid_edge_index, QIDS, UIDS, VIDS, CLICKS):
    raise NotImplementedError("write your pallas kernel here")


# speedup vs baseline: 8.4203x; 8.4203x over previous
import jax
import jax.numpy as jnp
from jax.experimental import pallas as pl
from jax.experimental.pallas import tpu as pltpu

_H = 4
_C = 32
_D = _H * _C
_SLOPE = 0.2
_NB = 1000
_EB = 2000


def _dense_body(x_ref, w_ref, asrc_ref, adst_ref, mask_ref,
                h_ref, ae_src_ref, ae_dst_ref):
    hb = jnp.dot(x_ref[...], w_ref[...], preferred_element_type=jnp.float32)
    h_ref[...] = hb
    m = mask_ref[...]
    ae_src_ref[...] = jnp.dot(hb * asrc_ref[...], m,
                              preferred_element_type=jnp.float32)
    ae_dst_ref[...] = jnp.dot(hb * adst_ref[...], m,
                              preferred_element_type=jnp.float32)


def _edge_body(edge_ref, h_ref, aes_ref, aed_ref, num_ref, den_ref):
    @pl.when(pl.program_id(0) == 0)
    def _init():
        num_ref[...] = jnp.zeros_like(num_ref)
        den_ref[...] = jnp.zeros_like(den_ref)

    eb = edge_ref.shape[2]

    def body(e, _):
        s = edge_ref[0, 0, e]
        d = edge_ref[0, 1, e]
        logit = aes_ref[pl.ds(s, 1), :] + aed_ref[pl.ds(d, 1), :]
        logit = jnp.where(logit > 0, logit, _SLOPE * logit)
        ex = jnp.exp(logit)
        den_ref[pl.ds(d, 1), :] = den_ref[pl.ds(d, 1), :] + ex
        num_ref[pl.ds(d, 1), :] = (num_ref[pl.ds(d, 1), :]
                                   + ex * h_ref[pl.ds(s, 1), :])
        return 0

    jax.lax.fori_loop(0, eb, body, 0)


def _final_body(num_ref, den_ref, b_ref, out_ref):
    out_ref[...] = jnp.maximum(
        num_ref[...] / (den_ref[...] + 1e-16) + b_ref[...], 0.0)


def _gather_body(qids_ref, uids_ref, vids_ref, clicks_ref,
                 pq_ref, pu_ref, vt_ref, ct_ref,
                 qe_ref, ue_ref, ve_ref, ce_ref):
    rows = qids_ref.shape[0]
    cols = qids_ref.shape[1]

    def body(i, _):
        r = i // cols
        c = i % cols
        q = qids_ref[r, c]
        u = uids_ref[r, c]
        v = vids_ref[r, c]
        k = clicks_ref[r, c]
        qe_ref[pl.ds(i, 1), :] = pq_ref[pl.ds(q, 1), :]
        ue_ref[pl.ds(i, 1), :] = pu_ref[pl.ds(u, 1), :]
        ve_ref[pl.ds(i, 1), :] = vt_ref[pl.ds(v, 1), :]
        ce_ref[pl.ds(i, 1), :] = ct_ref[pl.ds(k, 1), :]
        return 0

    jax.lax.fori_loop(0, rows * cols, body, 0)


def _gat(x, edge_index, W, a_src, a_dst, b):
    n = x.shape[0]
    e = edge_index.shape[1]
    nb = _NB if n % _NB == 0 else n
    eb = _EB if e % _EB == 0 else e
    mask = (jax.lax.broadcasted_iota(jnp.int32, (_D, _D), 0) // _C ==
            jax.lax.broadcasted_iota(jnp.int32, (_D, _D), 1) // _C
            ).astype(jnp.float32)
    h, aes, aed = pl.pallas_call(
        _dense_body,
        grid=(n // nb,),
        in_specs=[
            pl.BlockSpec((nb, _D), lambda i: (i, 0)),
            pl.BlockSpec((_D, _D), lambda i: (0, 0)),
            pl.BlockSpec((1, _D), lambda i: (0, 0)),
            pl.BlockSpec((1, _D), lambda i: (0, 0)),
            pl.BlockSpec((_D, _D), lambda i: (0, 0)),
        ],
        out_specs=[
            pl.BlockSpec((nb, _D), lambda i: (i, 0)),
            pl.BlockSpec((nb, _D), lambda i: (i, 0)),
            pl.BlockSpec((nb, _D), lambda i: (i, 0)),
        ],
        out_shape=[jax.ShapeDtypeStruct((n, _D), jnp.float32)] * 3,
    )(x, W, a_src.reshape(1, _D), a_dst.reshape(1, _D), mask)

    num, den = pl.pallas_call(
        _edge_body,
        grid=(e // eb,),
        in_specs=[
            pl.BlockSpec((1, 2, eb), lambda i: (i, 0, 0),
                         memory_space=pltpu.SMEM),
            pl.BlockSpec((n, _D), lambda i: (0, 0)),
            pl.BlockSpec((n, _D), lambda i: (0, 0)),
            pl.BlockSpec((n, _D), lambda i: (0, 0)),
        ],
        out_specs=[
            pl.BlockSpec((n, _D), lambda i: (0, 0)),
            pl.BlockSpec((n, _D), lambda i: (0, 0)),
        ],
        out_shape=[jax.ShapeDtypeStruct((n, _D), jnp.float32)] * 2,
    )(edge_index.reshape(2, e // eb, eb).transpose(1, 0, 2), h, aes, aed)

    out = pl.pallas_call(
        _final_body,
        grid=(n // nb,),
        in_specs=[
            pl.BlockSpec((nb, _D), lambda i: (i, 0)),
            pl.BlockSpec((nb, _D), lambda i: (i, 0)),
            pl.BlockSpec((1, _D), lambda i: (0, 0)),
        ],
        out_specs=pl.BlockSpec((nb, _D), lambda i: (i, 0)),
        out_shape=jax.ShapeDtypeStruct((n, _D), jnp.float32),
    )(num, den, b.reshape(1, _D))
    return out


def kernel(qid_table, uid_table, click_table, vid_table, pos_table,
           W_q, a_src_q, a_dst_q, b_q, W_u, a_src_u, a_dst_u, b_u,
           qid_edge_index, uid_edge_index, QIDS, UIDS, VIDS, CLICKS):
    pq = _gat(qid_table, qid_edge_index, W_q, a_src_q, a_dst_q, b_q)
    pu = _gat(uid_table, uid_edge_index, W_u, a_src_u, a_dst_u, b_u)

    bsz, seq = QIDS.shape
    rb = 128
    qe, ue, ve, ce = pl.pallas_call(
        _gather_body,
        grid=(bsz // rb,),
        in_specs=[
            pl.BlockSpec((rb, seq), lambda i: (i, 0),
                         memory_space=pltpu.SMEM),
            pl.BlockSpec((rb, seq), lambda i: (i, 0),
                         memory_space=pltpu.SMEM),
            pl.BlockSpec((rb, seq), lambda i: (i, 0),
                         memory_space=pltpu.SMEM),
            pl.BlockSpec((rb, seq), lambda i: (i, 0),
                         memory_space=pltpu.SMEM),
            pl.BlockSpec((pq.shape[0], _D), lambda i: (0, 0)),
            pl.BlockSpec((pu.shape[0], _D), lambda i: (0, 0)),
            pl.BlockSpec(vid_table.shape, lambda i: (0, 0)),
            pl.BlockSpec(click_table.shape, lambda i: (0, 0)),
        ],
        out_specs=[
            pl.BlockSpec((rb * seq, _D), lambda i: (i, 0)),
            pl.BlockSpec((rb * seq, _D), lambda i: (i, 0)),
            pl.BlockSpec((rb * seq, 16), lambda i: (i, 0)),
            pl.BlockSpec((rb * seq, 16), lambda i: (i, 0)),
        ],
        out_shape=[
            jax.ShapeDtypeStruct((bsz * seq, _D), jnp.float32),
            jax.ShapeDtypeStruct((bsz * seq, _D), jnp.float32),
            jax.ShapeDtypeStruct((bsz * seq, 16), jnp.float32),
            jax.ShapeDtypeStruct((bsz * seq, 16), jnp.float32),
        ],
    )(QIDS, UIDS, VIDS, CLICKS, pq, pu, vid_table, click_table)

    qid_embedding = qe.reshape(bsz, seq, _D)
    uid_embedding = ue.reshape(bsz, seq, _D)
    vid_embedding = ve.reshape(bsz, seq, 16)
    click_embedding = ce.reshape(bsz, seq, 16)
    pos_embedding = jnp.tile(pos_table[None, :, :], (bsz, seq // 10, 1))
    return (qid_embedding, uid_embedding, vid_embedding, click_embedding,
            pos_embedding)
